# baseline (speedup 1.0000x reference)
import functools

import jax
import jax.numpy as jnp
from jax import lax
from jax.experimental import pallas as pl
from jax.experimental.pallas import tpu as pltpu
from jax.experimental.pallas import tpu_sc as plsc

_NUM_EMBEDDINGS = 1000000
_D = 32
_F = 26
_B = 4096
_TOTAL = _F * _B

_NC = 2
_NS = 16
_L = 16
_NW = _NC * _NS
_B_PER_W = _TOTAL // _NW


def _make_gather():
    mesh = plsc.VectorSubcoreMesh(core_axis_name="c", subcore_axis_name="s")

    @functools.partial(
        pl.kernel,
        mesh=mesh,
        out_type=jax.ShapeDtypeStruct((_TOTAL, _D), jnp.float32),
        scratch_types=[
            pltpu.VMEM((_B_PER_W,), jnp.int32),
            pltpu.VMEM((_B_PER_W, _D), jnp.float32),
            pltpu.SemaphoreType.DMA,
        ],
        compiler_params=pltpu.CompilerParams(use_tc_tiling_on_sc=False),
    )
    def gather_kernel(values_hbm, table_hbm, out_hbm, idx_v, rows_v, sem):
        wid = lax.axis_index("s") * _NC + lax.axis_index("c")
        base = wid * _B_PER_W
        pltpu.sync_copy(values_hbm.at[pl.ds(base, _B_PER_W)], idx_v)

        def remap(i, carry):
            sl = pl.ds(i * _L, _L)
            idx_v[sl] = lax.rem(idx_v[sl], jnp.int32(_NUM_EMBEDDINGS))
            return carry

        lax.fori_loop(0, _B_PER_W // _L, remap, 0, unroll=8)

        pltpu.async_copy(table_hbm.at[idx_v], rows_v, sem).wait()
        pltpu.sync_copy(rows_v, out_hbm.at[pl.ds(base, _B_PER_W)])

    return gather_kernel


_gather = _make_gather()


def kernel(values, lengths, embedding_table):
    del lengths
    vals = values.astype(jnp.int32)
    out = _gather(vals, embedding_table)
    return out.reshape(_F, _B, _D)
